# baseline (speedup 1.0000x reference)
import jax
import jax.numpy as jnp
from jax.experimental import pallas as pl
from jax.experimental.pallas import tpu as pltpu

M_GRP = 16
C_TOT = 256
N_ITER = 5
EPS = 1e-3

_CORES = 2
_CHUNK_A = 7168
_CHUNK_B = 7168


def _moments_body(x_ref, m_ref):
    j = pl.program_id(1)

    @pl.when(j == 0)
    def _():
        m_ref[...] = jnp.zeros_like(m_ref)

    x = x_ref[...]
    gram = jax.lax.dot_general(
        x, x, (((0,), (0,)), ((), ())), preferred_element_type=jnp.float32)
    s = jnp.sum(x, axis=0, keepdims=True)
    m_ref[0:C_TOT, :] += gram
    m_ref[C_TOT:C_TOT + 8, :] += jnp.broadcast_to(s, (8, C_TOT))


def _whiten_matrix(m_ref, n):
    gram = m_ref[0, 0:C_TOT, :] + m_ref[1, 0:C_TOT, :]
    srow = m_ref[0, C_TOT:C_TOT + 1, :] + m_ref[1, C_TOT:C_TOT + 1, :]
    mu = srow * (1.0 / n)
    outer = jax.lax.dot_general(
        mu, mu, (((0,), (0,)), ((), ())), preferred_element_type=jnp.float32)
    cov = gram * (1.0 / n) - outer

    ri = jax.lax.broadcasted_iota(jnp.int32, (C_TOT, C_TOT), 0)
    ci = jax.lax.broadcasted_iota(jnp.int32, (C_TOT, C_TOT), 1)
    blk = ((ri // M_GRP) == (ci // M_GRP)).astype(jnp.float32)
    eye = (ri == ci).astype(jnp.float32)

    sigma = blk * ((1.0 - EPS) * cov) + EPS * eye
    diag_row = jnp.sum(sigma * eye, axis=0, keepdims=True)
    t_col = jnp.sum(blk * diag_row, axis=1, keepdims=True)
    sigma_n = sigma / t_col

    p = eye
    for _ in range(N_ITER):
        p3 = jnp.dot(jnp.dot(p, p, preferred_element_type=jnp.float32), p,
                     preferred_element_type=jnp.float32)
        p = 1.5 * p - 0.5 * jnp.dot(p3, sigma_n,
                                    preferred_element_type=jnp.float32)
    wm = p * jax.lax.rsqrt(t_col)
    return wm, mu


def _apply_body(n, m_ref, x_ref, o_ref, w_scr, mu_scr):
    j = pl.program_id(1)

    @pl.when(j == 0)
    def _():
        wm, mu = _whiten_matrix(m_ref, n)
        w_scr[...] = wm
        mu_scr[...] = jnp.broadcast_to(mu, (8, C_TOT))

    xc = x_ref[...] - mu_scr[0:1, :]
    o_ref[...] = jax.lax.dot_general(
        xc, w_scr[...], (((1,), (0,)), ((), ())),
        preferred_element_type=jnp.float32)


def kernel(inputs):
    b, w, h, c = inputs.shape
    n = b * w * h
    x = inputs.reshape(n, c)
    steps_a = n // (_CORES * _CHUNK_A)
    steps_b = n // (_CORES * _CHUNK_B)

    moments = pl.pallas_call(
        _moments_body,
        grid=(_CORES, steps_a),
        in_specs=[pl.BlockSpec((_CHUNK_A, C_TOT), lambda i, j: (i * steps_a + j, 0))],
        out_specs=pl.BlockSpec((None, C_TOT + 8, C_TOT), lambda i, j: (i, 0, 0)),
        out_shape=jax.ShapeDtypeStruct((_CORES, C_TOT + 8, C_TOT), jnp.float32),
        compiler_params=pltpu.CompilerParams(
            dimension_semantics=("parallel", "arbitrary"),
            vmem_limit_bytes=56 * 1024 * 1024),
        name="whiten_moments",
    )(x)

    out = pl.pallas_call(
        lambda *refs: _apply_body(float(n), *refs),
        grid=(_CORES, steps_b),
        in_specs=[pl.BlockSpec((_CORES, C_TOT + 8, C_TOT), lambda i, j: (0, 0, 0)),
                  pl.BlockSpec((_CHUNK_B, C_TOT), lambda i, j: (i * steps_b + j, 0))],
        out_specs=pl.BlockSpec((_CHUNK_B, C_TOT), lambda i, j: (i * steps_b + j, 0)),
        out_shape=jax.ShapeDtypeStruct((n, C_TOT), jnp.float32),
        scratch_shapes=[pltpu.VMEM((C_TOT, C_TOT), jnp.float32),
                        pltpu.VMEM((8, C_TOT), jnp.float32)],
        compiler_params=pltpu.CompilerParams(
            dimension_semantics=("parallel", "arbitrary")),
        name="whiten_apply",
    )(moments, x)

    return out.reshape(b, w, h, c)


# speedup vs baseline: 1.0262x; 1.0262x over previous
import jax
import jax.numpy as jnp
from jax.experimental import pallas as pl
from jax.experimental.pallas import tpu as pltpu

M_GRP = 16
C_TOT = 256
N_ITER = 5
EPS = 1e-3

_CORES = 2
_CHUNK_A = 14336
_CHUNK_B = 7168


def _moments_body(x_ref, m_ref):
    j = pl.program_id(1)

    @pl.when(j == 0)
    def _():
        m_ref[...] = jnp.zeros_like(m_ref)

    x = x_ref[...]
    gram = jax.lax.dot_general(
        x, x, (((0,), (0,)), ((), ())), preferred_element_type=jnp.float32)
    s = jnp.sum(x, axis=0, keepdims=True)
    m_ref[0:C_TOT, :] += gram
    m_ref[C_TOT:C_TOT + 8, :] += jnp.broadcast_to(s, (8, C_TOT))


def _whiten_matrix(m_ref, n):
    gram = m_ref[0, 0:C_TOT, :] + m_ref[1, 0:C_TOT, :]
    srow = m_ref[0, C_TOT:C_TOT + 1, :] + m_ref[1, C_TOT:C_TOT + 1, :]
    mu = srow * (1.0 / n)
    outer = jax.lax.dot_general(
        mu, mu, (((0,), (0,)), ((), ())), preferred_element_type=jnp.float32)
    cov = gram * (1.0 / n) - outer

    ri = jax.lax.broadcasted_iota(jnp.int32, (C_TOT, C_TOT), 0)
    ci = jax.lax.broadcasted_iota(jnp.int32, (C_TOT, C_TOT), 1)
    blk = ((ri // M_GRP) == (ci // M_GRP)).astype(jnp.float32)
    eye = (ri == ci).astype(jnp.float32)

    sigma = blk * ((1.0 - EPS) * cov) + EPS * eye
    diag_row = jnp.sum(sigma * eye, axis=0, keepdims=True)
    t_col = jnp.sum(blk * diag_row, axis=1, keepdims=True)
    sigma_n = sigma / t_col

    p = eye
    for _ in range(N_ITER):
        p3 = jnp.dot(jnp.dot(p, p, preferred_element_type=jnp.float32), p,
                     preferred_element_type=jnp.float32)
        p = 1.5 * p - 0.5 * jnp.dot(p3, sigma_n,
                                    preferred_element_type=jnp.float32)
    wm = p * jax.lax.rsqrt(t_col)
    return wm, mu


def _apply_body(n, m_ref, x_ref, o_ref, w_scr, mu_scr):
    j = pl.program_id(1)

    @pl.when(j == 0)
    def _():
        wm, mu = _whiten_matrix(m_ref, n)
        w_scr[...] = wm
        mu_scr[...] = jnp.broadcast_to(mu, (8, C_TOT))

    xc = x_ref[...] - mu_scr[0:1, :]
    o_ref[...] = jax.lax.dot_general(
        xc, w_scr[...], (((1,), (0,)), ((), ())),
        preferred_element_type=jnp.float32)


def kernel(inputs):
    b, w, h, c = inputs.shape
    n = b * w * h
    x = inputs.reshape(n, c)
    steps_a = n // (_CORES * _CHUNK_A)
    steps_b = n // (_CORES * _CHUNK_B)

    moments = pl.pallas_call(
        _moments_body,
        grid=(_CORES, steps_a),
        in_specs=[pl.BlockSpec((_CHUNK_A, C_TOT), lambda i, j: (i * steps_a + j, 0))],
        out_specs=pl.BlockSpec((None, C_TOT + 8, C_TOT), lambda i, j: (i, 0, 0)),
        out_shape=jax.ShapeDtypeStruct((_CORES, C_TOT + 8, C_TOT), jnp.float32),
        compiler_params=pltpu.CompilerParams(
            dimension_semantics=("parallel", "arbitrary")),
        name="whiten_moments",
    )(x)

    out = pl.pallas_call(
        lambda *refs: _apply_body(float(n), *refs),
        grid=(_CORES, steps_b),
        in_specs=[pl.BlockSpec((_CORES, C_TOT + 8, C_TOT), lambda i, j: (0, 0, 0)),
                  pl.BlockSpec((_CHUNK_B, C_TOT), lambda i, j: (i * steps_b + j, 0))],
        out_specs=pl.BlockSpec((_CHUNK_B, C_TOT), lambda i, j: (i * steps_b + j, 0)),
        out_shape=jax.ShapeDtypeStruct((n, C_TOT), jnp.float32),
        scratch_shapes=[pltpu.VMEM((C_TOT, C_TOT), jnp.float32),
                        pltpu.VMEM((8, C_TOT), jnp.float32)],
        compiler_params=pltpu.CompilerParams(
            dimension_semantics=("parallel", "arbitrary")),
        name="whiten_apply",
    )(moments, x)

    return out.reshape(b, w, h, c)


# speedup vs baseline: 1.0385x; 1.0120x over previous
import jax
import jax.numpy as jnp
from jax.experimental import pallas as pl
from jax.experimental.pallas import tpu as pltpu

M_GRP = 16
C_TOT = 256
N_ITER = 5
EPS = 1e-3

_CORES = 2
_CHUNK_A = 14336
_CHUNK_B = 12544


def _moments_body(x_ref, m_ref):
    j = pl.program_id(1)

    @pl.when(j == 0)
    def _():
        m_ref[...] = jnp.zeros_like(m_ref)

    x = x_ref[...]
    gram = jax.lax.dot_general(
        x, x, (((0,), (0,)), ((), ())), preferred_element_type=jnp.float32)
    s = jnp.sum(x, axis=0, keepdims=True)
    m_ref[0:C_TOT, :] += gram
    m_ref[C_TOT:C_TOT + 8, :] += jnp.broadcast_to(s, (8, C_TOT))


def _whiten_matrix(m_ref, n):
    gram = m_ref[0, 0:C_TOT, :] + m_ref[1, 0:C_TOT, :]
    srow = m_ref[0, C_TOT:C_TOT + 1, :] + m_ref[1, C_TOT:C_TOT + 1, :]
    mu = srow * (1.0 / n)
    outer = jax.lax.dot_general(
        mu, mu, (((0,), (0,)), ((), ())), preferred_element_type=jnp.float32)
    cov = gram * (1.0 / n) - outer

    ri = jax.lax.broadcasted_iota(jnp.int32, (C_TOT, C_TOT), 0)
    ci = jax.lax.broadcasted_iota(jnp.int32, (C_TOT, C_TOT), 1)
    blk = ((ri // M_GRP) == (ci // M_GRP)).astype(jnp.float32)
    eye = (ri == ci).astype(jnp.float32)

    sigma = blk * ((1.0 - EPS) * cov) + EPS * eye
    diag_row = jnp.sum(sigma * eye, axis=0, keepdims=True)
    t_col = jnp.sum(blk * diag_row, axis=1, keepdims=True)
    sigma_n = sigma / t_col

    p = eye
    for _ in range(N_ITER):
        p3 = jnp.dot(jnp.dot(p, p, preferred_element_type=jnp.float32), p,
                     preferred_element_type=jnp.float32)
        p = 1.5 * p - 0.5 * jnp.dot(p3, sigma_n,
                                    preferred_element_type=jnp.float32)
    wm = p * jax.lax.rsqrt(t_col)
    return wm, mu


def _apply_body(n, m_ref, x_ref, o_ref, w_scr, mu_scr):
    j = pl.program_id(1)

    @pl.when(j == 0)
    def _():
        wm, mu = _whiten_matrix(m_ref, n)
        w_scr[...] = wm
        mu_scr[...] = jnp.broadcast_to(mu, (8, C_TOT))

    xc = x_ref[...] - mu_scr[0:1, :]
    o_ref[...] = jax.lax.dot_general(
        xc, w_scr[...], (((1,), (0,)), ((), ())),
        preferred_element_type=jnp.float32)


def kernel(inputs):
    b, w, h, c = inputs.shape
    n = b * w * h
    x = inputs.reshape(n, c)
    steps_a = n // (_CORES * _CHUNK_A)
    steps_b = n // (_CORES * _CHUNK_B)

    moments = pl.pallas_call(
        _moments_body,
        grid=(_CORES, steps_a),
        in_specs=[pl.BlockSpec((_CHUNK_A, C_TOT), lambda i, j: (i * steps_a + j, 0))],
        out_specs=pl.BlockSpec((None, C_TOT + 8, C_TOT), lambda i, j: (i, 0, 0)),
        out_shape=jax.ShapeDtypeStruct((_CORES, C_TOT + 8, C_TOT), jnp.float32),
        compiler_params=pltpu.CompilerParams(
            dimension_semantics=("parallel", "arbitrary")),
        name="whiten_moments",
    )(x)

    out = pl.pallas_call(
        lambda *refs: _apply_body(float(n), *refs),
        grid=(_CORES, steps_b),
        in_specs=[pl.BlockSpec((_CORES, C_TOT + 8, C_TOT), lambda i, j: (0, 0, 0)),
                  pl.BlockSpec((_CHUNK_B, C_TOT), lambda i, j: (i * steps_b + j, 0))],
        out_specs=pl.BlockSpec((_CHUNK_B, C_TOT), lambda i, j: (i * steps_b + j, 0)),
        out_shape=jax.ShapeDtypeStruct((n, C_TOT), jnp.float32),
        scratch_shapes=[pltpu.VMEM((C_TOT, C_TOT), jnp.float32),
                        pltpu.VMEM((8, C_TOT), jnp.float32)],
        compiler_params=pltpu.CompilerParams(
            dimension_semantics=("parallel", "arbitrary"),
            vmem_limit_bytes=56 * 1024 * 1024),
        name="whiten_apply",
    )(moments, x)

    return out.reshape(b, w, h, c)
